# Initial kernel scaffold; baseline (speedup 1.0000x reference)
import functools
import math

import jax
import jax.numpy as jnp
from jax import lax
from jax.experimental import pallas as pl
from jax.experimental.pallas import tpu as pltpu
from jax.experimental.pallas import tpu_sc as plsc

_CUTOFF = 4.0
_NUM_BASIS = 16

_NC = 2
_NS = 16
_NW = _NC * _NS

_C = 2000


def _sc_gather(x, pos_t, edge_index, emb_x, emb_z):
    n = x.shape[0]
    e = edge_index.shape[1]
    assert e % _NW == 0
    ew = e // _NW
    assert ew % _C == 0
    full = ((n + _NW - 1) // _NW + 7) // 8 * 8
    last = n - (_NW - 1) * full
    assert 0 < last <= full

    mesh = plsc.VectorSubcoreMesh(
        core_axis_name="c", subcore_axis_name="s",
        num_cores=_NC, num_subcores=_NS)
    out_type = (
        jax.ShapeDtypeStruct((n, 8), jnp.float32),
        jax.ShapeDtypeStruct((n, 8), jnp.float32),
        jax.ShapeDtypeStruct((3, e), jnp.float32),
    )

    @functools.partial(pl.kernel, out_type=out_type, mesh=mesh)
    def sc_kernel(x_hbm, pos_hbm, ei_hbm, ex_hbm, ez_hbm,
                  hnx_hbm, hnz_hbm, dpl_hbm):
        wid = lax.axis_index("s") * _NC + lax.axis_index("c")

        def node_phase(idxb, rbx, rbz, sem):
            def do(base, cnt):
                pltpu.sync_copy(x_hbm.at[pl.ds(base, cnt)],
                                idxb.at[pl.ds(0, cnt)])
                cps = []
                off = 0
                while off < cnt:
                    k = min(128, cnt - off)
                    sl = pl.ds(off, k)
                    cps.append(pltpu.async_copy(
                        ex_hbm.at[idxb.at[sl]], rbx.at[sl], sem))
                    cps.append(pltpu.async_copy(
                        ez_hbm.at[idxb.at[sl]], rbz.at[sl], sem))
                    off += k
                for cp in cps:
                    cp.wait()
                pltpu.sync_copy(rbx.at[pl.ds(0, cnt)],
                                hnx_hbm.at[pl.ds(base, cnt)])
                pltpu.sync_copy(rbz.at[pl.ds(0, cnt)],
                                hnz_hbm.at[pl.ds(base, cnt)])

            @pl.when(wid < _NW - 1)
            def _():
                do(wid * full, full)

            @pl.when(wid == _NW - 1)
            def _():
                do((_NW - 1) * full, last)

        pl.run_scoped(
            node_phase,
            pltpu.VMEM((full,), jnp.int32),
            pltpu.VMEM((full, 8), jnp.float32),
            pltpu.VMEM((full, 8), jnp.float32),
            pltpu.SemaphoreType.DMA,
        )

        def edge_phase(plane, sidx, didx, dbuf):
            ebase = wid * ew
            for c in range(3):
                pltpu.sync_copy(pos_hbm.at[c], plane)

                def chunk(kk, _):
                    cb = ebase + kk * _C
                    pltpu.sync_copy(ei_hbm.at[0, pl.ds(cb, _C)], sidx)
                    pltpu.sync_copy(ei_hbm.at[1, pl.ds(cb, _C)], didx)

                    def step(i, _):
                        sl = pl.ds(i * 16, 16)
                        a = plsc.load_gather(plane, [didx[sl]])
                        b = plsc.load_gather(plane, [sidx[sl]])
                        dbuf[sl] = a - b
                        return 0

                    lax.fori_loop(0, _C // 16, step, 0)
                    pltpu.sync_copy(dbuf, dpl_hbm.at[c, pl.ds(cb, _C)])
                    return 0

                lax.fori_loop(0, ew // _C, chunk, 0)

        pl.run_scoped(
            edge_phase,
            pltpu.VMEM((n,), jnp.float32),
            pltpu.VMEM((_C,), jnp.int32),
            pltpu.VMEM((_C,), jnp.int32),
            pltpu.VMEM((_C,), jnp.float32),
        )

    return sc_kernel(x, pos_t, edge_index, emb_x, emb_z)


def _tc_bessel(dpl):
    e = dpl.shape[1]
    b = 3200
    assert e % b == 0

    def body(dref, h_ref, a_ref):
        d = dref[...]
        s = jnp.sum(d * d, axis=0, keepdims=True)
        r = jnp.sqrt(s)
        theta = r * (math.pi / _CUTOFF)
        sn = jnp.sin(theta)
        cs2 = 2.0 * jnp.cos(theta)
        g = math.sqrt(2.0 / _CUTOFF) / r
        rows = [sn]
        prev = jnp.zeros_like(sn)
        for _ in range(2, _NUM_BASIS + 1):
            nxt = cs2 * rows[-1] - prev
            prev = rows[-1]
            rows.append(nxt)
        h = jnp.concatenate(rows, axis=0) * g
        m = jnp.concatenate([h, d, jnp.zeros((5, d.shape[1]), jnp.float32)],
                            axis=0)
        ii = lax.broadcasted_iota(jnp.int32, (24, 24), 0)
        jj = lax.broadcasted_iota(jnp.int32, (24, 24), 1)
        ident = (ii == jj).astype(jnp.float32)
        t = lax.dot_general(m, ident, (((0,), (0,)), ((), ())),
                            preferred_element_type=jnp.float32)
        h_ref[...] = t[:, :_NUM_BASIS]
        a_ref[...] = t[:, _NUM_BASIS:_NUM_BASIS + 3]

    return pl.pallas_call(
        body,
        grid=(e // b,),
        in_specs=[pl.BlockSpec((3, b), lambda i: (0, i))],
        out_specs=[pl.BlockSpec((b, 16), lambda i: (i, 0)),
                   pl.BlockSpec((b, 3), lambda i: (i, 0))],
        out_shape=[jax.ShapeDtypeStruct((e, 16), jnp.float32),
                   jax.ShapeDtypeStruct((e, 3), jnp.float32)],
    )(dpl)


def kernel(x, positions, edge_index, embed_node_x, embed_node_z):
    pos_t = positions.T
    h_node_x, h_node_z, dpl = _sc_gather(
        x.astype(jnp.int32), pos_t, edge_index.astype(jnp.int32),
        embed_node_x, embed_node_z)
    h_edge, edge_attr = _tc_bessel(dpl)
    return (h_node_x, h_node_z, h_edge, edge_attr)


# speedup vs baseline: 5.1845x; 5.1845x over previous
import functools
import math

import jax
import jax.numpy as jnp
from jax import lax
from jax.experimental import pallas as pl
from jax.experimental.pallas import tpu as pltpu
from jax.experimental.pallas import tpu_sc as plsc

_CUTOFF = 4.0
_NUM_BASIS = 16

_NC = 2
_NS = 16
_NW = _NC * _NS

_C = 2000


def _sc_edge_diff(pos_flat, ei_flat, n):
    e = ei_flat.shape[0] // 2
    assert e % _NW == 0
    ew = e // _NW
    assert ew % _C == 0

    mesh = plsc.VectorSubcoreMesh(core_axis_name="c", subcore_axis_name="s")
    out_type = (
        jax.ShapeDtypeStruct((e,), jnp.float32),
        jax.ShapeDtypeStruct((e,), jnp.float32),
        jax.ShapeDtypeStruct((e,), jnp.float32),
    )
    scratch = [
        pltpu.VMEM((n,), jnp.float32),
        pltpu.VMEM((_C,), jnp.int32),
        pltpu.VMEM((_C,), jnp.int32),
        pltpu.VMEM((_C,), jnp.float32),
    ]

    @functools.partial(
        pl.kernel, out_type=out_type, mesh=mesh, scratch_types=scratch,
        compiler_params=pltpu.CompilerParams(needs_layout_passes=False))
    def sc_kernel(pos_hbm, ei_hbm, d0_hbm, d1_hbm, d2_hbm,
                  plane, sidx, didx, dbuf):
        wid = lax.axis_index("s") * _NC + lax.axis_index("c")

        for pofs, d_hbm in ((0, d0_hbm), (n, d1_hbm), (2 * n, d2_hbm)):
            pltpu.sync_copy(pos_hbm.at[pl.ds(pofs, n)], plane)

            def chunk(kk, _):
                cb = wid * ew + kk * _C
                pltpu.sync_copy(ei_hbm.at[pl.ds(cb, _C)], sidx)
                pltpu.sync_copy(ei_hbm.at[pl.ds(e + cb, _C)], didx)

                def step(i, _):
                    sl = pl.ds(i * 16, 16)
                    a = plsc.load_gather(plane, [didx[sl]])
                    b = plsc.load_gather(plane, [sidx[sl]])
                    dbuf[sl] = a - b
                    return 0

                lax.fori_loop(0, _C // 16, step, 0)
                pltpu.sync_copy(dbuf, d_hbm.at[pl.ds(cb, _C)])
                return 0

            lax.fori_loop(0, ew // _C, chunk, 0)

    return sc_kernel(pos_flat, ei_flat)


def _tc_node_embed(x, emb2):
    n = x.shape[0]
    b = 4000
    assert n % b == 0
    nb = n // b
    xr = x.reshape(nb, 1, b)
    s = emb2.shape[0]

    def body(x_ref, e_ref, hx_ref, hz_ref):
        xv = x_ref[...].reshape(1, b)
        sp = lax.broadcasted_iota(jnp.int32, (s, 1), 0)
        onehot = (xv == sp).astype(jnp.float32)
        t = lax.dot_general(onehot, e_ref[...], (((0,), (0,)), ((), ())),
                            preferred_element_type=jnp.float32)
        hx_ref[...] = t[:, :8]
        hz_ref[...] = t[:, 8:]

    return pl.pallas_call(
        body,
        grid=(nb,),
        in_specs=[pl.BlockSpec((1, 1, b), lambda i: (i, 0, 0)),
                  pl.BlockSpec((s, 16), lambda i: (0, 0))],
        out_specs=[pl.BlockSpec((b, 8), lambda i: (i, 0)),
                   pl.BlockSpec((b, 8), lambda i: (i, 0))],
        out_shape=[jax.ShapeDtypeStruct((n, 8), jnp.float32),
                   jax.ShapeDtypeStruct((n, 8), jnp.float32)],
    )(xr, emb2)


def _tc_bessel(d0, d1, d2):
    e = d0.shape[0]
    b = 3200
    assert e % b == 0
    nb = e // b
    dr = [d.reshape(nb, 1, b) for d in (d0, d1, d2)]

    def body(xr, yr, zr, h_ref, a_ref):
        dx = xr[...].reshape(1, b)
        dy = yr[...].reshape(1, b)
        dz = zr[...].reshape(1, b)
        r = jnp.sqrt(dx * dx + dy * dy + dz * dz)
        theta = r * (math.pi / _CUTOFF)
        sn = jnp.sin(theta)
        cs2 = 2.0 * jnp.cos(theta)
        g = math.sqrt(2.0 / _CUTOFF) / r
        rows = [sn]
        prev = jnp.zeros_like(sn)
        for _ in range(2, _NUM_BASIS + 1):
            nxt = cs2 * rows[-1] - prev
            prev = rows[-1]
            rows.append(nxt)
        h = jnp.concatenate(rows, axis=0) * g
        m = jnp.concatenate([h, dx, dy, dz,
                             jnp.zeros((5, b), jnp.float32)], axis=0)
        ii = lax.broadcasted_iota(jnp.int32, (24, 24), 0)
        jj = lax.broadcasted_iota(jnp.int32, (24, 24), 1)
        ident = (ii == jj).astype(jnp.float32)
        t = lax.dot_general(m, ident, (((0,), (0,)), ((), ())),
                            preferred_element_type=jnp.float32)
        h_ref[...] = t[:, :_NUM_BASIS]
        a_ref[...] = t[:, _NUM_BASIS:_NUM_BASIS + 3]

    return pl.pallas_call(
        body,
        grid=(nb,),
        in_specs=[pl.BlockSpec((1, 1, b), lambda i: (i, 0, 0))] * 3,
        out_specs=[pl.BlockSpec((b, 16), lambda i: (i, 0)),
                   pl.BlockSpec((b, 3), lambda i: (i, 0))],
        out_shape=[jax.ShapeDtypeStruct((e, 16), jnp.float32),
                   jax.ShapeDtypeStruct((e, 3), jnp.float32)],
    )(*dr)


def kernel(x, positions, edge_index, embed_node_x, embed_node_z):
    n = positions.shape[0]
    pos_flat = positions.T.reshape(-1)
    ei_flat = edge_index.astype(jnp.int32).reshape(-1)
    emb2 = jnp.concatenate([embed_node_x, embed_node_z], axis=1)
    h_node_x, h_node_z = _tc_node_embed(x.astype(jnp.int32), emb2)
    d0, d1, d2 = _sc_edge_diff(pos_flat, ei_flat, n)
    h_edge, edge_attr = _tc_bessel(d0, d1, d2)
    return (h_node_x, h_node_z, h_edge, edge_attr)


# speedup vs baseline: 15.0101x; 2.8952x over previous
import functools
import math

import jax
import jax.numpy as jnp
from jax import lax
from jax.experimental import pallas as pl
from jax.experimental.pallas import tpu as pltpu
from jax.experimental.pallas import tpu_sc as plsc

_CUTOFF = 4.0
_NUM_BASIS = 16

_NC = 2
_NS = 16
_NW = _NC * _NS

_C = 2000


def _sc_edge_diff(pos_flat, ei_flat, n):
    e = ei_flat.shape[0] // 2
    assert e % _NW == 0
    ew = e // _NW
    assert ew % _C == 0

    mesh = plsc.VectorSubcoreMesh(core_axis_name="c", subcore_axis_name="s")
    out_type = (
        jax.ShapeDtypeStruct((e,), jnp.float32),
        jax.ShapeDtypeStruct((e,), jnp.float32),
        jax.ShapeDtypeStruct((e,), jnp.float32),
    )
    scratch = [
        pltpu.VMEM((n,), jnp.float32),
        pltpu.VMEM((_C,), jnp.int32),
        pltpu.VMEM((_C,), jnp.int32),
        pltpu.VMEM((_C,), jnp.float32),
    ]

    @functools.partial(
        pl.kernel, out_type=out_type, mesh=mesh, scratch_types=scratch,
        compiler_params=pltpu.CompilerParams(needs_layout_passes=False))
    def sc_kernel(pos_hbm, ei_hbm, d0_hbm, d1_hbm, d2_hbm,
                  plane, sidx, didx, dbuf):
        wid = lax.axis_index("s") * _NC + lax.axis_index("c")

        for pofs, d_hbm in ((0, d0_hbm), (n, d1_hbm), (2 * n, d2_hbm)):
            pltpu.sync_copy(pos_hbm.at[pl.ds(pofs, n)], plane)

            def chunk(kk, _):
                cb = wid * ew + kk * _C
                pltpu.sync_copy(ei_hbm.at[pl.ds(cb, _C)], sidx)
                pltpu.sync_copy(ei_hbm.at[pl.ds(e + cb, _C)], didx)

                def step(i, _):
                    sl = pl.ds(i * 16, 16)
                    a = plsc.load_gather(plane, [didx[sl]])
                    b = plsc.load_gather(plane, [sidx[sl]])
                    dbuf[sl] = a - b
                    return 0

                lax.fori_loop(0, _C // 16, step, 0)
                pltpu.sync_copy(dbuf, d_hbm.at[pl.ds(cb, _C)])
                return 0

            lax.fori_loop(0, ew // _C, chunk, 0)

    return sc_kernel(pos_flat, ei_flat)


def _tc_node_embed(x, emb2t):
    n = x.shape[0]
    b = 3200
    assert n % b == 0
    nb = n // b
    xr = x.reshape(nb, 1, b)
    s = emb2t.shape[1]

    def body(x_ref, e_ref, hx_ref, hz_ref):
        xv = x_ref[...].reshape(1, b)
        sp = lax.broadcasted_iota(jnp.int32, (s, 1), 0)
        onehot = (xv == sp).astype(jnp.float32)
        t = lax.dot_general(e_ref[...], onehot, (((1,), (0,)), ((), ())),
                            preferred_element_type=jnp.float32)
        hx_ref[...] = t[:8, :]
        hz_ref[...] = t[8:, :]

    return pl.pallas_call(
        body,
        grid=(nb,),
        in_specs=[pl.BlockSpec((1, 1, b), lambda i: (i, 0, 0)),
                  pl.BlockSpec((16, s), lambda i: (0, 0))],
        out_specs=[pl.BlockSpec((8, b), lambda i: (0, i)),
                   pl.BlockSpec((8, b), lambda i: (0, i))],
        out_shape=[jax.ShapeDtypeStruct((8, n), jnp.float32),
                   jax.ShapeDtypeStruct((8, n), jnp.float32)],
    )(xr, emb2t)


def _tc_bessel(d0, d1, d2):
    e = d0.shape[0]
    b = 3200
    assert e % b == 0
    nb = e // b
    dr = [d.reshape(nb, 1, b) for d in (d0, d1, d2)]

    def body(xr, yr, zr, h_ref, a_ref):
        dx = xr[...].reshape(1, b)
        dy = yr[...].reshape(1, b)
        dz = zr[...].reshape(1, b)
        r = jnp.sqrt(dx * dx + dy * dy + dz * dz)
        theta = r * (math.pi / _CUTOFF)
        sn = jnp.sin(theta)
        cs2 = 2.0 * jnp.cos(theta)
        g = math.sqrt(2.0 / _CUTOFF) / r
        rows = [sn]
        prev = jnp.zeros_like(sn)
        for _ in range(2, _NUM_BASIS + 1):
            nxt = cs2 * rows[-1] - prev
            prev = rows[-1]
            rows.append(nxt)
        h_ref[...] = jnp.concatenate(rows, axis=0) * g
        a_ref[...] = jnp.concatenate([dx, dy, dz], axis=0)

    return pl.pallas_call(
        body,
        grid=(nb,),
        in_specs=[pl.BlockSpec((1, 1, b), lambda i: (i, 0, 0))] * 3,
        out_specs=[pl.BlockSpec((_NUM_BASIS, b), lambda i: (0, i)),
                   pl.BlockSpec((3, b), lambda i: (0, i))],
        out_shape=[jax.ShapeDtypeStruct((_NUM_BASIS, e), jnp.float32),
                   jax.ShapeDtypeStruct((3, e), jnp.float32)],
    )(*dr)


def kernel(x, positions, edge_index, embed_node_x, embed_node_z):
    n = positions.shape[0]
    pos_flat = positions.T.reshape(-1)
    ei_flat = edge_index.astype(jnp.int32).reshape(-1)
    emb2t = jnp.concatenate([embed_node_x, embed_node_z], axis=1).T
    npad = -n % 3200
    xpad = jnp.pad(x.astype(jnp.int32), (0, npad))
    hx, hz = _tc_node_embed(xpad, emb2t)
    hx = lax.slice(hx, (0, 0), (8, n))
    hz = lax.slice(hz, (0, 0), (8, n))
    d0, d1, d2 = _sc_edge_diff(pos_flat, ei_flat, n)
    h16, a3 = _tc_bessel(d0, d1, d2)
    return (hx.T, hz.T, h16.T, a3.T)
